# baseline (speedup 1.0000x reference)
import functools

import jax
import jax.numpy as jnp
from jax import lax
from jax.experimental import pallas as pl
from jax.experimental.pallas import tpu as pltpu
from jax.experimental.pallas import tpu_sc as plsc

NTOKEN = 100000
D = 64
B = 16384
L = 50
N = B * L
NC = 2
NS = 16
NW = NC * NS
BATCH_W = B // NW
CHUNK = 2 * L
NSTEPS = BATCH_W // 2
LANES = 16

_mesh = plsc.VectorSubcoreMesh(core_axis_name="c", subcore_axis_name="s")


@functools.partial(
    pl.kernel,
    out_type=(
        jax.ShapeDtypeStruct((B, L, D), jnp.float32),
        jax.ShapeDtypeStruct((B, L, D), jnp.float32),
        jax.ShapeDtypeStruct((NW * LANES,), jnp.float32),
    ),
    mesh=_mesh,
    compiler_params=pltpu.CompilerParams(
        use_tc_tiling_on_sc=True, needs_layout_passes=False),
    scratch_types=[
        [pltpu.VMEM((128,), jnp.int32)] * 2,
        [pltpu.VMEM((128,), jnp.int32)] * 2,
        [pltpu.VMEM((128,), jnp.int32)] * 2,
        [pltpu.VMEM((CHUNK, 128), jnp.float32)] * 2,
        [pltpu.VMEM((CHUNK, 128), jnp.float32)] * 2,
        [pltpu.VMEM((CHUNK, D), jnp.float32)] * 2,
        [pltpu.VMEM((CHUNK, D), jnp.float32)] * 2,
        pltpu.VMEM((LANES,), jnp.float32),
        [pltpu.SemaphoreType.DMA] * 2,
        [pltpu.SemaphoreType.DMA] * 2,
        [pltpu.SemaphoreType.DMA] * 2,
    ],
)
def _sc_gather(idx_hbm, enc_hbm, glv_hbm, out_e, out_g, out_p,
               raw_slots, ih_slots, p64_slots, e_slots, g_slots,
               es_slots, gs_slots, acc_v, isems, gsems, wsems):
    wid = lax.axis_index("s") * NC + lax.axis_index("c")
    crow0 = wid * NSTEPS
    brow0 = wid * BATCH_W

    def fire_raw(k2, s2):
        pltpu.async_copy(idx_hbm.at[crow0 + k2], raw_slots[s2], isems[s2])

    def wait_raw(s2):
        pltpu.make_async_copy(idx_hbm.at[0], raw_slots[s2], isems[s2]).wait()

    def prep_idx(s2):
        raw, ih, p64 = raw_slots[s2], ih_slots[s2], p64_slots[s2]
        for t in range(7):
            v = raw[pl.ds(16 * t, LANES)]
            ih[pl.ds(16 * t, LANES)] = v >> 1
            p64[pl.ds(16 * t, LANES)] = (v & 1) << 6

    def fire_gather(s2):
        idx_ref = ih_slots[s2].at[pl.ds(0, CHUNK)]
        pltpu.async_copy(enc_hbm.at[idx_ref], e_slots[s2], gsems[s2])
        pltpu.async_copy(glv_hbm.at[idx_ref], g_slots[s2], gsems[s2])

    def wait_gather(s2):
        idx_ref = ih_slots[s2].at[pl.ds(0, CHUNK)]
        pltpu.make_async_copy(enc_hbm.at[idx_ref], e_slots[s2], gsems[s2]).wait()
        pltpu.make_async_copy(glv_hbm.at[idx_ref], g_slots[s2], gsems[s2]).wait()

    def fire_write(k, s2):
        b0 = brow0 + 2 * k
        pltpu.async_copy(es_slots[s2].at[pl.ds(0, L)], out_e.at[b0], wsems[s2])
        pltpu.async_copy(es_slots[s2].at[pl.ds(L, L)], out_e.at[b0 + 1], wsems[s2])
        pltpu.async_copy(gs_slots[s2].at[pl.ds(0, L)], out_g.at[b0], wsems[s2])
        pltpu.async_copy(gs_slots[s2].at[pl.ds(L, L)], out_g.at[b0 + 1], wsems[s2])

    def wait_write(s2):
        for ref in (es_slots[s2], es_slots[s2], gs_slots[s2], gs_slots[s2]):
            pltpu.make_async_copy(ref.at[pl.ds(0, L)], out_e.at[0], wsems[s2]).wait()

    def compute(s2, accs):
        e_v, g_v = e_slots[s2], g_slots[s2]
        es_v, gs_v = es_slots[s2], gs_slots[s2]
        p64 = p64_slots[s2]
        a0, a1 = accs
        for gi in range(7):
            r0 = 16 * gi if gi < 6 else 84
            rows = lax.iota(jnp.int32, LANES) + r0
            par = p64[pl.ds(r0, LANES)]
            fresh = (lax.iota(jnp.int32, LANES) >= 12) if gi == 6 else None

            def col_body(ci, ab, rows=rows, par=par, fresh=fresh):
                b0v, b1v = ab
                c = 2 * ci
                cvec = jnp.zeros((LANES,), jnp.int32) + c
                ve0 = plsc.load_gather(e_v, [rows, par + c])
                vg0 = plsc.load_gather(g_v, [rows, par + c])
                plsc.store_scatter(es_v, [rows, cvec], ve0)
                plsc.store_scatter(gs_v, [rows, cvec], vg0)
                ve1 = plsc.load_gather(e_v, [rows, par + (c + 1)])
                vg1 = plsc.load_gather(g_v, [rows, par + (c + 1)])
                plsc.store_scatter(es_v, [rows, cvec + 1], ve1)
                plsc.store_scatter(gs_v, [rows, cvec + 1], vg1)
                d0 = ve0 - vg0
                d1 = ve1 - vg1
                if fresh is not None:
                    zero = jnp.zeros((LANES,), jnp.float32)
                    d0 = jnp.where(fresh, d0, zero)
                    d1 = jnp.where(fresh, d1, zero)
                return (b0v + d0 * d0, b1v + d1 * d1)

            a0, a1 = lax.fori_loop(0, D // 2, col_body, (a0, a1))
        return (a0, a1)

    def service(k, s2, accs, *, drain_w=True, do_next=True, do_raw=True):
        wait_gather(s2)
        if drain_w:
            wait_write(s2)
        accs = compute(s2, accs)
        fire_write(k, s2)
        if do_next:
            wait_raw(s2)
            prep_idx(s2)
            if do_raw:
                fire_raw(k + 4, s2)
            fire_gather(s2)
        return accs

    fire_raw(0, 0)
    fire_raw(1, 1)
    wait_raw(0)
    prep_idx(0)
    fire_gather(0)
    fire_raw(2, 0)
    wait_raw(1)
    prep_idx(1)
    fire_gather(1)
    fire_raw(3, 1)

    zero = jnp.zeros((LANES,), jnp.float32)
    accs = (zero, zero)
    accs = service(0, 0, accs, drain_w=False)
    accs = service(1, 1, accs, drain_w=False)

    def group_body(g, accs):
        k = 2 * g
        accs = service(k, 0, accs)
        accs = service(k + 1, 1, accs)
        return accs

    accs = lax.fori_loop(1, NSTEPS // 2 - 2, group_body, accs)

    k = NSTEPS - 4
    accs = service(k, 0, accs, do_raw=False)
    accs = service(k + 1, 1, accs, do_raw=False)
    accs = service(k + 2, 0, accs, do_next=False)
    accs = service(k + 3, 1, accs, do_next=False)
    wait_write(0)
    wait_write(1)

    a0, a1 = accs
    acc_v[...] = a0 + a1
    pltpu.sync_copy(acc_v, out_p.at[pl.ds(wid * LANES, LANES)])


def _tc_sum_body(p_ref, o_ref):
    o_ref[0, 0] = jnp.sum(p_ref[...]) * jnp.float32(1.0 / (N * D))


_tc_sum = pl.pallas_call(
    _tc_sum_body,
    out_shape=jax.ShapeDtypeStruct((1, 1), jnp.float32),
    out_specs=pl.BlockSpec(memory_space=pltpu.SMEM),
)


def kernel(input, encoder_weight, glove_weight):
    idx = input.reshape(N // CHUNK, CHUNK).astype(jnp.int32)
    idx = jnp.pad(idx, ((0, 0), (0, 128 - CHUNK)))
    enc2 = encoder_weight.reshape(NTOKEN // 2, 2 * D)
    glv2 = glove_weight.reshape(NTOKEN // 2, 2 * D)
    emb, emb_glove, parts = _sc_gather(idx, enc2, glv2)
    glove_loss = _tc_sum(parts.reshape(4, 128))[0, 0]
    return (emb, emb_glove, glove_loss)


# speedup vs baseline: 3.2557x; 3.2557x over previous
import functools

import jax
import jax.numpy as jnp
from jax import lax
from jax.experimental import pallas as pl
from jax.experimental.pallas import tpu as pltpu
from jax.experimental.pallas import tpu_sc as plsc

NTOKEN = 100000
D = 64
B = 16384
L = 50
N = B * L
NC = 2
NS = 16
NW = NC * NS
BATCH_W = B // NW
CHUNK = 2 * L
NSTEPS = BATCH_W // 2
LANES = 16

_mesh = plsc.VectorSubcoreMesh(core_axis_name="c", subcore_axis_name="s")


@functools.partial(
    pl.kernel,
    out_type=(
        jax.ShapeDtypeStruct((B, L, D), jnp.float32),
        jax.ShapeDtypeStruct((B, L, D), jnp.float32),
        jax.ShapeDtypeStruct((NW * LANES,), jnp.float32),
    ),
    mesh=_mesh,
    compiler_params=pltpu.CompilerParams(
        use_tc_tiling_on_sc=True, needs_layout_passes=False),
    scratch_types=[
        [pltpu.VMEM((128,), jnp.int32)] * 4,
        [pltpu.VMEM((CHUNK, 128), jnp.float32)] * 2,
        [pltpu.VMEM((CHUNK, 128), jnp.float32)] * 2,
        [pltpu.VMEM((CHUNK, D), jnp.float32)] * 2,
        [pltpu.VMEM((CHUNK, D), jnp.float32)] * 2,
        pltpu.VMEM((LANES,), jnp.float32),
        [pltpu.SemaphoreType.DMA] * 4,
        [pltpu.SemaphoreType.DMA] * 2,
        [pltpu.SemaphoreType.DMA] * 2,
    ],
)
def _sc_gather(idx_hbm, enc_hbm, glv_hbm, out_e, out_g, out_p,
               idx_slots, e_slots, g_slots, es_slots, gs_slots,
               acc_v, isems, gsems, wsems):
    wid = lax.axis_index("s") * NC + lax.axis_index("c")
    crow0 = wid * NSTEPS
    brow0 = wid * BATCH_W

    def fire_idx(k2, i2):
        pltpu.async_copy(idx_hbm.at[crow0 + k2], idx_slots[i2], isems[i2])

    def wait_idx(i2):
        pltpu.make_async_copy(idx_hbm.at[0], idx_slots[i2], isems[i2]).wait()

    def fire_gather(i2, s2):
        idx_ref = idx_slots[i2].at[pl.ds(0, CHUNK)]
        pltpu.async_copy(enc_hbm.at[idx_ref], e_slots[s2], gsems[s2])
        pltpu.async_copy(glv_hbm.at[idx_ref], g_slots[s2], gsems[s2])

    def wait_gather(s2):
        idx_ref = idx_slots[0].at[pl.ds(0, CHUNK)]
        pltpu.make_async_copy(enc_hbm.at[idx_ref], e_slots[s2], gsems[s2]).wait()
        pltpu.make_async_copy(glv_hbm.at[idx_ref], g_slots[s2], gsems[s2]).wait()

    def fire_write(k, s2):
        b0 = brow0 + 2 * k
        pltpu.async_copy(es_slots[s2].at[pl.ds(0, L)], out_e.at[b0], wsems[s2])
        pltpu.async_copy(es_slots[s2].at[pl.ds(L, L)], out_e.at[b0 + 1], wsems[s2])
        pltpu.async_copy(gs_slots[s2].at[pl.ds(0, L)], out_g.at[b0], wsems[s2])
        pltpu.async_copy(gs_slots[s2].at[pl.ds(L, L)], out_g.at[b0 + 1], wsems[s2])

    def wait_write(s2):
        for _ in range(4):
            pltpu.make_async_copy(es_slots[s2].at[pl.ds(0, L)],
                                  out_e.at[0], wsems[s2]).wait()

    def compute(s2, accs):
        e_v, g_v = e_slots[s2], g_slots[s2]
        es_v, gs_v = es_slots[s2], gs_slots[s2]

        def pair_body(p, ab):
            a0, a1, a2, a3 = ab
            for half in range(2):
                r = 2 * p + half
                for j in range(4):
                    ve = e_v[r, pl.ds(16 * j, LANES)]
                    vg = g_v[r, pl.ds(16 * j, LANES)]
                    es_v[r, pl.ds(16 * j, LANES)] = ve
                    gs_v[r, pl.ds(16 * j, LANES)] = vg
                    d = ve - vg
                    if j == 0:
                        a0 += d * d
                    elif j == 1:
                        a1 += d * d
                    elif j == 2:
                        a2 += d * d
                    else:
                        a3 += d * d
            return (a0, a1, a2, a3)

        return lax.fori_loop(0, CHUNK // 2, pair_body, accs)

    def service(k, b, accs, *, drain_w=True, do_idx=True, do_next=True):
        s2 = b % 2
        wait_gather(s2)
        if do_idx:
            fire_idx(k + 4, b)
        if drain_w:
            wait_write(s2)
        accs = compute(s2, accs)
        fire_write(k, s2)
        if do_next:
            i2 = (b + 2) % 4
            wait_idx(i2)
            fire_gather(i2, s2)
        return accs

    for c in range(4):
        fire_idx(c, c)
    wait_idx(0)
    fire_gather(0, 0)
    wait_idx(1)
    fire_gather(1, 1)

    zero = jnp.zeros((LANES,), jnp.float32)
    accs = (zero, zero, zero, zero)
    accs = service(0, 0, accs, drain_w=False)
    accs = service(1, 1, accs, drain_w=False)
    accs = service(2, 2, accs)
    accs = service(3, 3, accs)

    def group_body(g, accs):
        k = 4 * g
        for b in range(4):
            accs = service(k + b, b, accs)
        return accs

    accs = lax.fori_loop(1, NSTEPS // 4 - 1, group_body, accs)

    k = NSTEPS - 4
    accs = service(k, 0, accs, do_idx=False)
    accs = service(k + 1, 1, accs, do_idx=False)
    accs = service(k + 2, 2, accs, do_idx=False, do_next=False)
    accs = service(k + 3, 3, accs, do_idx=False, do_next=False)
    wait_write(0)
    wait_write(1)

    a0, a1, a2, a3 = accs
    acc_v[...] = (a0 + a1) + (a2 + a3)
    pltpu.sync_copy(acc_v, out_p.at[pl.ds(wid * LANES, LANES)])


def _tc_sum_body(p_ref, o_ref):
    o_ref[0, 0] = jnp.sum(p_ref[...]) * jnp.float32(1.0 / (N * D))


_tc_sum = pl.pallas_call(
    _tc_sum_body,
    out_shape=jax.ShapeDtypeStruct((1, 1), jnp.float32),
    out_specs=pl.BlockSpec(memory_space=pltpu.SMEM),
)


def _lines(table):
    return jnp.concatenate([table, jnp.roll(table, -1, axis=0)], axis=1)


def kernel(input, encoder_weight, glove_weight):
    idx = input.reshape(N // CHUNK, CHUNK).astype(jnp.int32)
    idx = jnp.pad(idx, ((0, 0), (0, 128 - CHUNK)))
    emb, emb_glove, parts = _sc_gather(idx, _lines(encoder_weight),
                                       _lines(glove_weight))
    glove_loss = _tc_sum(parts.reshape(4, 128))[0, 0]
    return (emb, emb_glove, glove_loss)
